# Initial kernel scaffold; baseline (speedup 1.0000x reference)
import jax
import jax.numpy as jnp
from jax.experimental import pallas as pl


def kernel(x, edge_index, W, a):
    raise NotImplementedError("write your pallas kernel here")


# speedup vs baseline: 1.8214x; 1.8214x over previous
import functools

import jax
import jax.numpy as jnp
from jax import lax
from jax.experimental import pallas as pl
from jax.experimental.pallas import tpu as pltpu
from jax.experimental.pallas import tpu_sc as plsc

N_NODES = 10000
N_EDGES = 320000
D = 128
DH = D // 2
E_REAL = N_EDGES + N_NODES

NC, NS = 2, 16
NPAD = 10240
ROWS_PER_TILE = NPAD // NS
BLK = 128
SB = 9
NSUP = 18
NBLK = SB * NSUP
E_CHUNK = NBLK * BLK
EPAD = NS * E_CHUNK
DUMMY = N_NODES


def _mm_body(x_ref, wt_ref, a2_ref, act2_ref, asrc_ref, adst_ref):
    act = jnp.dot(x_ref[...], wt_ref[...], preferred_element_type=jnp.float32)
    act2_ref[0] = act[:, :DH]
    act2_ref[1] = act[:, DH:]
    al = jnp.dot(act, a2_ref[...], preferred_element_type=jnp.float32)
    lane = lax.broadcasted_iota(jnp.int32, (act.shape[0], 16), 1)
    adst = al[:, 0]
    asrc = al[:, 1]
    asrc_ref[...] = jnp.where(lane == 0, asrc[:, None], 0.0)
    adst_ref[...] = jnp.where(lane == 0, adst[:, None], 0.0)


def _matmul(x_pad, wt, a2):
    blk = 512
    return pl.pallas_call(
        _mm_body,
        grid=(NPAD // blk,),
        in_specs=[
            pl.BlockSpec((blk, D), lambda i: (i, 0)),
            pl.BlockSpec((D, D), lambda i: (0, 0)),
            pl.BlockSpec((D, 8), lambda i: (0, 0)),
        ],
        out_specs=[
            pl.BlockSpec((2, blk, DH), lambda i: (0, i, 0)),
            pl.BlockSpec((blk, 16), lambda i: (i, 0)),
            pl.BlockSpec((blk, 16), lambda i: (i, 0)),
        ],
        out_shape=[
            jax.ShapeDtypeStruct((2, NPAD, DH), jnp.float32),
            jax.ShapeDtypeStruct((NPAD, 16), jnp.float32),
            jax.ShapeDtypeStruct((NPAD, 16), jnp.float32),
        ],
    )(x_pad, wt, a2)


def _sc_body(glh_hbm, src_hbm, dst_hbm, asrc_hbm, adst_hbm,
             msg_hbm, w_hbm, aggp_hbm, denp_hbm,
             src_sb, srcg_sb, dst_sb, w_sb, rows_v, wrow_v, asrc_blk, adst_blk,
             agg_sh, den_sh, sem):
    c = lax.axis_index("c")
    s = lax.axis_index("s")

    def _zero_row(e, carry):
        for v in range(DH // 16):
            rows_v[e, pl.ds(v * 16, 16)] = jnp.zeros((16,), jnp.float32)
        wrow_v[e, :] = jnp.zeros((16,), jnp.float32)
        return carry

    lax.fori_loop(0, BLK, _zero_row, 0)
    for b in range(ROWS_PER_TILE // BLK):
        r0 = s * ROWS_PER_TILE + b * BLK
        pltpu.sync_copy(rows_v, agg_sh.at[pl.ds(r0, BLK)])
        pltpu.sync_copy(wrow_v, den_sh.at[pl.ds(r0, BLK)])
    plsc.subcore_barrier()

    ebase0 = s * E_CHUNK
    lane = lax.iota(jnp.int32, 16)
    zeros16i = jnp.zeros((16,), jnp.int32)
    coff = jnp.full((16,), c, jnp.int32) * jnp.int32(NPAD)

    def _block(b, k):
        j = k * SB + b

        pltpu.async_copy(glh_hbm.at[srcg_sb.at[b]], rows_v, sem).wait()

        cp_as = pltpu.async_copy(asrc_hbm.at[src_sb.at[b]], asrc_blk, sem)
        cp_ad = pltpu.async_copy(adst_hbm.at[dst_sb.at[b]], adst_blk, sem)

        pltpu.sync_copy(
            rows_v,
            msg_hbm.at[pl.ds(ebase0 + j * BLK, BLK), c],
        )

        cp_as.wait()
        cp_ad.wait()

        for g in range(8):
            e16 = lane + g * 16
            z = plsc.load_gather(adst_blk, [e16, zeros16i]) + plsc.load_gather(
                asrc_blk, [e16, zeros16i]
            )
            z = jnp.where(z >= 0.0, z, z * jnp.float32(0.01))
            w_sb[b, pl.ds(g * 16, 16)] = jnp.exp(z)

        def _edge(e, carry2):
            wb = plsc.load_gather(
                w_sb,
                [jnp.full((16,), b, jnp.int32), jnp.full((16,), e, jnp.int32)],
            )
            for v in range(DH // 16):
                rows_v[e, pl.ds(v * 16, 16)] = rows_v[e, pl.ds(v * 16, 16)] * wb
            wrow_v[e, :] = jnp.where(lane == 0, wb, jnp.float32(0.0))
            return carry2

        lax.fori_loop(0, BLK, _edge, 0)

        pltpu.sync_copy(rows_v, agg_sh.at[dst_sb.at[b]], add=True)
        pltpu.sync_copy(wrow_v, den_sh.at[dst_sb.at[b]], add=True)
        return k

    def _super(k, carry):
        pltpu.sync_copy(src_hbm.at[s, pl.ds(k * SB, SB)], src_sb)
        pltpu.sync_copy(dst_hbm.at[s, pl.ds(k * SB, SB)], dst_sb)
        for b in range(SB):
            for g in range(8):
                srcg_sb[b, pl.ds(g * 16, 16)] = (
                    src_sb[b, pl.ds(g * 16, 16)] + coff
                )
        lax.fori_loop(0, SB, _block, k)

        pltpu.sync_copy(w_sb, w_hbm.at[s, pl.ds(k * SB, SB)])
        return carry

    lax.fori_loop(0, NSUP, _super, 0)

    plsc.subcore_barrier()
    for b in range(ROWS_PER_TILE // BLK):
        r0 = s * ROWS_PER_TILE + b * BLK
        pltpu.sync_copy(agg_sh.at[pl.ds(r0, BLK)], rows_v)
        pltpu.sync_copy(rows_v, aggp_hbm.at[pl.ds(r0, BLK), c])

    for b in range(ROWS_PER_TILE // BLK):
        r0 = s * ROWS_PER_TILE + b * BLK
        pltpu.sync_copy(den_sh.at[pl.ds(r0, BLK)], wrow_v)
        pltpu.sync_copy(wrow_v, denp_hbm.at[pl.ds(r0, BLK)])


_sc_call = functools.partial(
    pl.kernel,
    out_type=[
        jax.ShapeDtypeStruct((EPAD, 2, DH), jnp.float32),
        jax.ShapeDtypeStruct((NS, NBLK, BLK), jnp.float32),
        jax.ShapeDtypeStruct((NPAD, 2, DH), jnp.float32),
        jax.ShapeDtypeStruct((NPAD, 16), jnp.float32),
    ],
    mesh=plsc.VectorSubcoreMesh(
        core_axis_name="c", subcore_axis_name="s", num_cores=NC, num_subcores=NS
    ),
    compiler_params=pltpu.CompilerParams(
        use_tc_tiling_on_sc=False, needs_layout_passes=False
    ),
    scratch_types=[
        pltpu.VMEM((SB, BLK), jnp.int32),
        pltpu.VMEM((SB, BLK), jnp.int32),
        pltpu.VMEM((SB, BLK), jnp.int32),
        pltpu.VMEM((SB, BLK), jnp.float32),
        pltpu.VMEM((BLK, DH), jnp.float32),
        pltpu.VMEM((BLK, 16), jnp.float32),
        pltpu.VMEM((BLK, 16), jnp.float32),
        pltpu.VMEM((BLK, 16), jnp.float32),
        pltpu.VMEM_SHARED((NPAD, DH), jnp.float32),
        pltpu.VMEM_SHARED((NPAD, 16), jnp.float32),
        pltpu.SemaphoreType.DMA,
    ],
)


def _comb_body(aggp_ref, denp_ref, agg_ref, den_ref):
    d = denp_ref[:, 0]
    safe = jnp.where(d == 0.0, jnp.float32(1.0), d)
    agg_ref[...] = aggp_ref[...] / safe[:, None]
    den_ref[...] = safe


def _combine(aggp, denp):
    blk = 512
    return pl.pallas_call(
        _comb_body,
        grid=(NPAD // blk,),
        in_specs=[
            pl.BlockSpec((blk, D), lambda i: (i, 0)),
            pl.BlockSpec((blk, 16), lambda i: (i, 0)),
        ],
        out_specs=[
            pl.BlockSpec((blk, D), lambda i: (i, 0)),
            pl.BlockSpec((blk,), lambda i: (i,)),
        ],
        out_shape=[
            jax.ShapeDtypeStruct((NPAD, D), jnp.float32),
            jax.ShapeDtypeStruct((NPAD,), jnp.float32),
        ],
    )(aggp, denp)


def kernel(x, edge_index, W, a):
    x_pad = jnp.zeros((NPAD, D), jnp.float32).at[:N_NODES].set(x)
    a2 = jnp.zeros((D, 8), jnp.float32)
    a2 = a2.at[:, 0].set(a[:D]).at[:, 1].set(a[D:])
    act2, asrc16, adst16 = _matmul(x_pad, W.T, a2)
    glh = act2.reshape(2 * NPAD, DH)

    src = edge_index[0].astype(jnp.int32)
    dst = edge_index[1].astype(jnp.int32)
    loop = jnp.arange(N_NODES, dtype=jnp.int32)
    npad_e = EPAD - E_REAL
    src_full = jnp.concatenate([src, loop, jnp.zeros((npad_e,), jnp.int32)])
    dst_full = jnp.concatenate([dst, loop, jnp.full((npad_e,), DUMMY, jnp.int32)])
    src3d = src_full.reshape(NS, NBLK, BLK)
    dst3d = dst_full.reshape(NS, NBLK, BLK)

    msg3, w3d, aggp3, denp = _sc_call(_sc_body)(
        glh, src3d, dst3d, asrc16, adst16
    )
    msgs = msg3.reshape(EPAD, D)
    agg_full, den_full = _combine(aggp3.reshape(NPAD, D), denp)

    return (
        agg_full[:N_NODES],
        w3d.reshape(EPAD)[:E_REAL],
        den_full[:N_NODES],
        msgs[:E_REAL],
    )


# speedup vs baseline: 1.9978x; 1.0968x over previous
import functools

import jax
import jax.numpy as jnp
from jax import lax
from jax.experimental import pallas as pl
from jax.experimental.pallas import tpu as pltpu
from jax.experimental.pallas import tpu_sc as plsc

N_NODES = 10000
N_EDGES = 320000
D = 128
DH = D // 2
DW = 80
E_REAL = N_EDGES + N_NODES

NC, NS = 2, 16
NPAD = 10240
ROWS_PER_TILE = NPAD // NS
BLK = 128
SB = 6
NSUP = 27
NBLK = SB * NSUP
E_CHUNK = NBLK * BLK
EPAD = NS * E_CHUNK
DUMMY = N_NODES


def _mm_body(x_ref, wt_ref, a2_ref, act2_ref, adst_ref):
    act = jnp.dot(x_ref[...], wt_ref[...], preferred_element_type=jnp.float32,
                  precision=lax.Precision.HIGHEST)
    al = jnp.dot(act, a2_ref[...], preferred_element_type=jnp.float32,
                 precision=lax.Precision.HIGHEST)
    lane = lax.broadcasted_iota(jnp.int32, (act.shape[0], 16), 1)
    adst = al[:, 0]
    asrc = al[:, 1]
    asrc16 = jnp.where(lane == 0, asrc[:, None], 0.0)
    act2_ref[0, :, :DH] = act[:, :DH]
    act2_ref[0, :, DH:] = asrc16
    act2_ref[1, :, :DH] = act[:, DH:]
    act2_ref[1, :, DH:] = asrc16
    adst_ref[...] = jnp.where(lane == 0, adst[:, None], 0.0)


def _matmul(x_pad, wt, a2):
    blk = 512
    return pl.pallas_call(
        _mm_body,
        grid=(NPAD // blk,),
        in_specs=[
            pl.BlockSpec((blk, D), lambda i: (i, 0)),
            pl.BlockSpec((D, D), lambda i: (0, 0)),
            pl.BlockSpec((D, 8), lambda i: (0, 0)),
        ],
        out_specs=[
            pl.BlockSpec((2, blk, DW), lambda i: (0, i, 0)),
            pl.BlockSpec((blk, 16), lambda i: (i, 0)),
        ],
        out_shape=[
            jax.ShapeDtypeStruct((2, NPAD, DW), jnp.float32),
            jax.ShapeDtypeStruct((NPAD, 16), jnp.float32),
        ],
    )(x_pad, wt, a2)


def _sc_body(glh_hbm, src_hbm, dst_hbm, adst_hbm,
             msg_hbm, w_hbm, aggp_hbm, denp_hbm,
             srcg_sb, dst_sb, w_sb, rows_a, rows_b, adst_a, adst_b,
             acc_sh, sem_a, sem_b):
    c = lax.axis_index("c")
    s = lax.axis_index("s")

    def _zero_row(e, carry):
        for v in range(DW // 16):
            rows_a[e, pl.ds(v * 16, 16)] = jnp.zeros((16,), jnp.float32)
        return carry

    lax.fori_loop(0, BLK, _zero_row, 0)
    for b in range(ROWS_PER_TILE // BLK):
        pltpu.sync_copy(rows_a, acc_sh.at[pl.ds(s * ROWS_PER_TILE + b * BLK, BLK)])
    plsc.subcore_barrier()

    ebase0 = s * E_CHUNK
    lane = lax.iota(jnp.int32, 16)
    zeros16i = jnp.zeros((16,), jnp.int32)
    col64 = jnp.full((16,), DH, jnp.int32)
    coff = jnp.full((16,), c, jnp.int32) * jnp.int32(NPAD)
    bufs = ((rows_a, adst_a, sem_a), (rows_b, adst_b, sem_b))

    def _fire(b, rows_v, adst_v, sem):
        return (
            pltpu.async_copy(glh_hbm.at[srcg_sb.at[b]], rows_v, sem),
            pltpu.async_copy(adst_hbm.at[dst_sb.at[b]], adst_v, sem),
        )

    def _super(k, carry):
        pltpu.sync_copy(src_hbm.at[s, pl.ds(k * SB, SB)], srcg_sb)
        pltpu.sync_copy(dst_hbm.at[s, pl.ds(k * SB, SB)], dst_sb)
        for b in range(SB):
            for g in range(8):
                srcg_sb[b, pl.ds(g * 16, 16)] = (
                    srcg_sb[b, pl.ds(g * 16, 16)] + coff
                )

        cps = _fire(0, *bufs[0])
        for b in range(SB):
            rows_v, adst_v, _ = bufs[b % 2]
            if b + 1 < SB:
                nxt = _fire(b + 1, *bufs[(b + 1) % 2])
            cps[0].wait()
            cps[1].wait()
            if b + 1 < SB:
                cps = nxt

            pltpu.sync_copy(
                rows_v.at[:, pl.ds(0, DH)],
                msg_hbm.at[pl.ds(ebase0 + (k * SB + b) * BLK, BLK), c],
            )

            for g in range(8):
                e16 = lane + g * 16
                z = plsc.load_gather(adst_v, [e16, zeros16i]) + plsc.load_gather(
                    rows_v, [e16, col64]
                )
                z = jnp.where(z >= 0.0, z, z * jnp.float32(0.01))
                w_sb[b, pl.ds(g * 16, 16)] = jnp.exp(z)

            def _edge(e, carry2):
                wb = plsc.load_gather(
                    w_sb,
                    [jnp.full((16,), b, jnp.int32), jnp.full((16,), e, jnp.int32)],
                )
                for v in range(DH // 16):
                    rows_v[e, pl.ds(v * 16, 16)] = (
                        rows_v[e, pl.ds(v * 16, 16)] * wb
                    )
                rows_v[e, pl.ds(DH, 16)] = jnp.where(lane == 0, wb, jnp.float32(0.0))
                return carry2

            lax.fori_loop(0, BLK, _edge, 0)

            pltpu.sync_copy(rows_v, acc_sh.at[dst_sb.at[b]], add=True)

        pltpu.sync_copy(w_sb, w_hbm.at[s, pl.ds(k * SB, SB)])
        return carry

    lax.fori_loop(0, NSUP, _super, 0)

    plsc.subcore_barrier()
    for b in range(ROWS_PER_TILE // BLK):
        r0 = s * ROWS_PER_TILE + b * BLK
        pltpu.sync_copy(acc_sh.at[pl.ds(r0, BLK)], rows_a)
        pltpu.sync_copy(rows_a.at[:, pl.ds(0, DH)], aggp_hbm.at[pl.ds(r0, BLK), c])
        pltpu.sync_copy(rows_a.at[:, pl.ds(DH, 16)], denp_hbm.at[pl.ds(r0, BLK)])


_sc_call = functools.partial(
    pl.kernel,
    out_type=[
        jax.ShapeDtypeStruct((EPAD, 2, DH), jnp.float32),
        jax.ShapeDtypeStruct((NS, NBLK, BLK), jnp.float32),
        jax.ShapeDtypeStruct((NPAD, 2, DH), jnp.float32),
        jax.ShapeDtypeStruct((NPAD, 16), jnp.float32),
    ],
    mesh=plsc.VectorSubcoreMesh(
        core_axis_name="c", subcore_axis_name="s", num_cores=NC, num_subcores=NS
    ),
    compiler_params=pltpu.CompilerParams(
        use_tc_tiling_on_sc=False, needs_layout_passes=False
    ),
    scratch_types=[
        pltpu.VMEM((SB, BLK), jnp.int32),
        pltpu.VMEM((SB, BLK), jnp.int32),
        pltpu.VMEM((SB, BLK), jnp.float32),
        pltpu.VMEM((BLK, DW), jnp.float32),
        pltpu.VMEM((BLK, DW), jnp.float32),
        pltpu.VMEM((BLK, 16), jnp.float32),
        pltpu.VMEM((BLK, 16), jnp.float32),
        pltpu.VMEM_SHARED((NPAD, DW), jnp.float32),
        pltpu.SemaphoreType.DMA,
        pltpu.SemaphoreType.DMA,
    ],
)


def _comb_body(aggp_ref, denp_ref, agg_ref, den_ref):
    d = denp_ref[:, 0]
    safe = jnp.where(d == 0.0, jnp.float32(1.0), d)
    agg_ref[...] = aggp_ref[...] / safe[:, None]
    den_ref[...] = safe


def _combine(aggp, denp):
    blk = 512
    return pl.pallas_call(
        _comb_body,
        grid=(NPAD // blk,),
        in_specs=[
            pl.BlockSpec((blk, D), lambda i: (i, 0)),
            pl.BlockSpec((blk, 16), lambda i: (i, 0)),
        ],
        out_specs=[
            pl.BlockSpec((blk, D), lambda i: (i, 0)),
            pl.BlockSpec((blk,), lambda i: (i,)),
        ],
        out_shape=[
            jax.ShapeDtypeStruct((NPAD, D), jnp.float32),
            jax.ShapeDtypeStruct((NPAD,), jnp.float32),
        ],
    )(aggp, denp)


def kernel(x, edge_index, W, a):
    x_pad = jnp.zeros((NPAD, D), jnp.float32).at[:N_NODES].set(x)
    a2 = jnp.zeros((D, 8), jnp.float32)
    a2 = a2.at[:, 0].set(a[:D]).at[:, 1].set(a[D:])
    act2, adst16 = _matmul(x_pad, W.T, a2)
    glh = act2.reshape(2 * NPAD, DW)

    src = edge_index[0].astype(jnp.int32)
    dst = edge_index[1].astype(jnp.int32)
    loop = jnp.arange(N_NODES, dtype=jnp.int32)
    npad_e = EPAD - E_REAL
    src_full = jnp.concatenate([src, loop, jnp.zeros((npad_e,), jnp.int32)])
    dst_full = jnp.concatenate([dst, loop, jnp.full((npad_e,), DUMMY, jnp.int32)])
    src3d = src_full.reshape(NS, NBLK, BLK)
    dst3d = dst_full.reshape(NS, NBLK, BLK)

    msg3, w3d, aggp3, denp = _sc_call(_sc_body)(glh, src3d, dst3d, adst16)
    msgs = msg3.reshape(EPAD, D)
    agg_full, den_full = _combine(aggp3.reshape(NPAD, D), denp)

    return (
        agg_full[:N_NODES],
        w3d.reshape(EPAD)[:E_REAL],
        den_full[:N_NODES],
        msgs[:E_REAL],
    )


# speedup vs baseline: 2.0968x; 1.0496x over previous
import functools

import jax
import jax.numpy as jnp
from jax import lax
from jax.experimental import pallas as pl
from jax.experimental.pallas import tpu as pltpu
from jax.experimental.pallas import tpu_sc as plsc

N_NODES = 10000
N_EDGES = 320000
D = 128
DH = D // 2
DW = 80
E_REAL = N_EDGES + N_NODES

NC, NS = 2, 16
NPAD = 10240
ROWS_PER_TILE = NPAD // NS
BLK = 128
SB = 6
NSUP = 27
NBLK = SB * NSUP
E_CHUNK = NBLK * BLK
EPAD = NS * E_CHUNK
DUMMY = N_NODES


def _mm_body(x_ref, wt_ref, a2_ref, act2_ref, adst_ref):
    act = jnp.dot(x_ref[...], wt_ref[...], preferred_element_type=jnp.float32)
    al = jnp.dot(act, a2_ref[...], preferred_element_type=jnp.float32)
    lane = lax.broadcasted_iota(jnp.int32, (act.shape[0], 16), 1)
    adst = al[:, 0]
    asrc = al[:, 1]
    asrc16 = jnp.where(lane == 0, asrc[:, None], 0.0)
    act2_ref[0, :, :DH] = act[:, :DH]
    act2_ref[0, :, DH:] = asrc16
    act2_ref[1, :, :DH] = act[:, DH:]
    act2_ref[1, :, DH:] = asrc16
    adst_ref[...] = jnp.where(lane == 0, adst[:, None], 0.0)


def _matmul(x_pad, wt, a2):
    blk = 512
    return pl.pallas_call(
        _mm_body,
        grid=(NPAD // blk,),
        in_specs=[
            pl.BlockSpec((blk, D), lambda i: (i, 0)),
            pl.BlockSpec((D, D), lambda i: (0, 0)),
            pl.BlockSpec((D, 8), lambda i: (0, 0)),
        ],
        out_specs=[
            pl.BlockSpec((2, blk, DW), lambda i: (0, i, 0)),
            pl.BlockSpec((blk, 16), lambda i: (i, 0)),
        ],
        out_shape=[
            jax.ShapeDtypeStruct((2, NPAD, DW), jnp.float32),
            jax.ShapeDtypeStruct((NPAD, 16), jnp.float32),
        ],
    )(x_pad, wt, a2)


def _sc_body(glh_hbm, src_hbm, dst_hbm, adst_hbm,
             msg_hbm, w_hbm, aggp_hbm, denp_hbm,
             srcg_sb, dst_sb, w_sb, rows_a, rows_b, adst_a, adst_b,
             acc_sh, sem_a, sem_b):
    c = lax.axis_index("c")
    s = lax.axis_index("s")

    def _zero_row(e, carry):
        for v in range(DW // 16):
            rows_a[e, pl.ds(v * 16, 16)] = jnp.zeros((16,), jnp.float32)
        return carry

    lax.fori_loop(0, BLK, _zero_row, 0)
    for b in range(ROWS_PER_TILE // BLK):
        pltpu.sync_copy(rows_a, acc_sh.at[pl.ds(s * ROWS_PER_TILE + b * BLK, BLK)])
    plsc.subcore_barrier()

    ebase0 = s * E_CHUNK
    lane = lax.iota(jnp.int32, 16)
    zeros16i = jnp.zeros((16,), jnp.int32)
    col64 = jnp.full((16,), DH, jnp.int32)
    coff = jnp.full((16,), c, jnp.int32) * jnp.int32(NPAD)
    bufs = ((rows_a, adst_a, sem_a), (rows_b, adst_b, sem_b))

    def _fire(b, rows_v, adst_v, sem):
        return (
            pltpu.async_copy(glh_hbm.at[srcg_sb.at[b]], rows_v, sem),
            pltpu.async_copy(adst_hbm.at[dst_sb.at[b]], adst_v, sem),
        )

    def _super(k, carry):
        pltpu.sync_copy(src_hbm.at[s, pl.ds(k * SB, SB)], srcg_sb)
        pltpu.sync_copy(dst_hbm.at[s, pl.ds(k * SB, SB)], dst_sb)
        for b in range(SB):
            for g in range(8):
                srcg_sb[b, pl.ds(g * 16, 16)] = (
                    srcg_sb[b, pl.ds(g * 16, 16)] + coff
                )

        cps = _fire(0, *bufs[0])
        for b in range(SB):
            rows_v, adst_v, _ = bufs[b % 2]
            if b + 1 < SB:
                nxt = _fire(b + 1, *bufs[(b + 1) % 2])
            cps[0].wait()
            cps[1].wait()
            if b + 1 < SB:
                cps = nxt

            pltpu.sync_copy(
                rows_v.at[:, pl.ds(0, DH)],
                msg_hbm.at[pl.ds(ebase0 + (k * SB + b) * BLK, BLK), c],
            )

            for g in range(8):
                e16 = lane + g * 16
                z = plsc.load_gather(adst_v, [e16, zeros16i]) + plsc.load_gather(
                    rows_v, [e16, col64]
                )
                z = jnp.where(z >= 0.0, z, z * jnp.float32(0.01))
                w_sb[b, pl.ds(g * 16, 16)] = jnp.exp(z)

            brow = jnp.full((16,), b, jnp.int32)

            @plsc.parallel_loop(0, BLK, 1, unroll=4)
            def _edge(e):
                wb = plsc.load_gather(w_sb, [brow, jnp.full((16,), e, jnp.int32)])
                for v in range(DH // 16):
                    rows_v[e, pl.ds(v * 16, 16)] = (
                        rows_v[e, pl.ds(v * 16, 16)] * wb
                    )
                rows_v[e, pl.ds(DH, 16)] = jnp.where(lane == 0, wb, jnp.float32(0.0))

            pltpu.sync_copy(rows_v, acc_sh.at[dst_sb.at[b]], add=True)

        pltpu.sync_copy(w_sb, w_hbm.at[s, pl.ds(k * SB, SB)])
        return carry

    lax.fori_loop(0, NSUP, _super, 0)

    plsc.subcore_barrier()
    for b in range(ROWS_PER_TILE // BLK):
        r0 = s * ROWS_PER_TILE + b * BLK
        pltpu.sync_copy(acc_sh.at[pl.ds(r0, BLK)], rows_a)
        pltpu.sync_copy(rows_a.at[:, pl.ds(0, DH)], aggp_hbm.at[pl.ds(r0, BLK), c])
        pltpu.sync_copy(rows_a.at[:, pl.ds(DH, 16)], denp_hbm.at[pl.ds(r0, BLK)])


_sc_call = functools.partial(
    pl.kernel,
    out_type=[
        jax.ShapeDtypeStruct((EPAD, 2, DH), jnp.float32),
        jax.ShapeDtypeStruct((NS, NBLK, BLK), jnp.float32),
        jax.ShapeDtypeStruct((NPAD, 2, DH), jnp.float32),
        jax.ShapeDtypeStruct((NPAD, 16), jnp.float32),
    ],
    mesh=plsc.VectorSubcoreMesh(
        core_axis_name="c", subcore_axis_name="s", num_cores=NC, num_subcores=NS
    ),
    compiler_params=pltpu.CompilerParams(
        use_tc_tiling_on_sc=False, needs_layout_passes=False
    ),
    scratch_types=[
        pltpu.VMEM((SB, BLK), jnp.int32),
        pltpu.VMEM((SB, BLK), jnp.int32),
        pltpu.VMEM((SB, BLK), jnp.float32),
        pltpu.VMEM((BLK, DW), jnp.float32),
        pltpu.VMEM((BLK, DW), jnp.float32),
        pltpu.VMEM((BLK, 16), jnp.float32),
        pltpu.VMEM((BLK, 16), jnp.float32),
        pltpu.VMEM_SHARED((NPAD, DW), jnp.float32),
        pltpu.SemaphoreType.DMA,
        pltpu.SemaphoreType.DMA,
    ],
)


def _comb_body(aggp_ref, denp_ref, agg_ref, den_ref):
    d = denp_ref[:, 0]
    safe = jnp.where(d == 0.0, jnp.float32(1.0), d)
    agg_ref[...] = aggp_ref[...] / safe[:, None]
    den_ref[...] = safe


def _combine(aggp, denp):
    blk = 512
    return pl.pallas_call(
        _comb_body,
        grid=(NPAD // blk,),
        in_specs=[
            pl.BlockSpec((blk, D), lambda i: (i, 0)),
            pl.BlockSpec((blk, 16), lambda i: (i, 0)),
        ],
        out_specs=[
            pl.BlockSpec((blk, D), lambda i: (i, 0)),
            pl.BlockSpec((blk,), lambda i: (i,)),
        ],
        out_shape=[
            jax.ShapeDtypeStruct((NPAD, D), jnp.float32),
            jax.ShapeDtypeStruct((NPAD,), jnp.float32),
        ],
    )(aggp, denp)


def kernel(x, edge_index, W, a):
    x_pad = jnp.zeros((NPAD, D), jnp.float32).at[:N_NODES].set(x)
    a2 = jnp.zeros((D, 8), jnp.float32)
    a2 = a2.at[:, 0].set(a[:D]).at[:, 1].set(a[D:])
    act2, adst16 = _matmul(x_pad, W.T, a2)
    glh = act2.reshape(2 * NPAD, DW)

    src = edge_index[0].astype(jnp.int32)
    dst = edge_index[1].astype(jnp.int32)
    loop = jnp.arange(N_NODES, dtype=jnp.int32)
    npad_e = EPAD - E_REAL
    src_full = jnp.concatenate([src, loop, jnp.zeros((npad_e,), jnp.int32)])
    dst_full = jnp.concatenate([dst, loop, jnp.full((npad_e,), DUMMY, jnp.int32)])
    src3d = src_full.reshape(NS, NBLK, BLK)
    dst3d = dst_full.reshape(NS, NBLK, BLK)

    msg3, w3d, aggp3, denp = _sc_call(_sc_body)(glh, src3d, dst3d, adst16)
    msgs = msg3.reshape(EPAD, D)
    agg_full, den_full = _combine(aggp3.reshape(NPAD, D), denp)

    return (
        agg_full[:N_NODES],
        w3d.reshape(EPAD)[:E_REAL],
        den_full[:N_NODES],
        msgs[:E_REAL],
    )


# speedup vs baseline: 5.8747x; 2.8017x over previous
import functools

import jax
import jax.numpy as jnp
from jax import lax
from jax.experimental import pallas as pl
from jax.experimental.pallas import tpu as pltpu
from jax.experimental.pallas import tpu_sc as plsc

N_NODES = 10000
N_EDGES = 320000
D = 128
DH = D // 2
E_REAL = N_EDGES + N_NODES

NC, NS = 2, 16
NPAD = 10240
ROWS_PER_TILE = NPAD // NS
BLK = 128
SB = 6
NSUP = 27
NBLK = SB * NSUP
E_CHUNK = NBLK * BLK
EPAD = NS * E_CHUNK
DUMMY = N_NODES


def _mm_body(x_ref, wt_ref, a2_ref, act2_ref, asrc_ref, adst_ref):
    act = jnp.dot(x_ref[...], wt_ref[...], preferred_element_type=jnp.float32)
    al = jnp.dot(act, a2_ref[...], preferred_element_type=jnp.float32)
    lane = lax.broadcasted_iota(jnp.int32, (act.shape[0], 16), 1)
    asrc16 = jnp.where(lane == 0, al[:, 1][:, None], 0.0)
    act2_ref[0] = act[:, :DH]
    act2_ref[1] = act[:, DH:]
    asrc_ref[0] = asrc16
    asrc_ref[1] = asrc16
    adst_ref[...] = jnp.where(lane == 0, al[:, 0][:, None], 0.0)


def _matmul(x, wt, a2):
    blk = 400
    return pl.pallas_call(
        _mm_body,
        grid=(N_NODES // blk,),
        in_specs=[
            pl.BlockSpec((blk, D), lambda i: (i, 0)),
            pl.BlockSpec((D, D), lambda i: (0, 0)),
            pl.BlockSpec((D, 8), lambda i: (0, 0)),
        ],
        out_specs=[
            pl.BlockSpec((2, blk, DH), lambda i: (0, i, 0)),
            pl.BlockSpec((2, blk, 16), lambda i: (0, i, 0)),
            pl.BlockSpec((blk, 16), lambda i: (i, 0)),
        ],
        out_shape=[
            jax.ShapeDtypeStruct((2, NPAD, DH), jnp.float32),
            jax.ShapeDtypeStruct((2, NPAD, 16), jnp.float32),
            jax.ShapeDtypeStruct((NPAD, 16), jnp.float32),
        ],
    )(x, wt, a2)


def _sc_body(glh_hbm, src_hbm, dst_hbm, mpos_hbm, asrc_hbm, adst_hbm,
             msg_hbm, w_hbm, aggp_hbm, denp_hbm,
             srcg_sb, dst_sb, midx_sb, w_sb, rows_a, rows_b, wrow_v,
             asrc_blk, adst_blk, agg_sh, den_sh, sem_a, sem_b, sem_c):
    c = lax.axis_index("c")
    s = lax.axis_index("s")

    def _zero_row(e, carry):
        for v in range(DH // 16):
            rows_a[e, pl.ds(v * 16, 16)] = jnp.zeros((16,), jnp.float32)
        wrow_v[e, :] = jnp.zeros((16,), jnp.float32)
        return carry

    lax.fori_loop(0, BLK, _zero_row, 0)
    for b in range(ROWS_PER_TILE // BLK):
        r0 = s * ROWS_PER_TILE + b * BLK
        pltpu.sync_copy(rows_a, agg_sh.at[pl.ds(r0, BLK)])
        pltpu.sync_copy(wrow_v, den_sh.at[pl.ds(r0, BLK)])
    plsc.subcore_barrier()

    lane = lax.iota(jnp.int32, 16)
    zeros16i = jnp.zeros((16,), jnp.int32)
    coff = jnp.full((16,), c, jnp.int32) * jnp.int32(NPAD)
    cvec = jnp.full((16,), c, jnp.int32)
    bufs = ((rows_a, sem_a), (rows_b, sem_b))

    def _fire(b, rows_v, sem):
        return pltpu.async_copy(glh_hbm.at[srcg_sb.at[b]], rows_v, sem)

    def _super(k, carry):
        pltpu.sync_copy(src_hbm.at[s, pl.ds(k * SB, SB)], srcg_sb)
        pltpu.sync_copy(dst_hbm.at[s, pl.ds(k * SB, SB)], dst_sb)
        pltpu.sync_copy(mpos_hbm.at[s, pl.ds(k * SB, SB)], midx_sb)
        for b in range(SB):
            for g in range(8):
                srcg_sb[b, pl.ds(g * 16, 16)] = (
                    srcg_sb[b, pl.ds(g * 16, 16)] + coff
                )
                midx_sb[b, pl.ds(g * 16, 16)] = (
                    midx_sb[b, pl.ds(g * 16, 16)] + cvec
                )

        cp = _fire(0, *bufs[0])
        for b in range(SB):
            rows_v, _ = bufs[b % 2]
            if b + 1 < SB:
                nxt = _fire(b + 1, *bufs[(b + 1) % 2])
            cp_as = pltpu.async_copy(asrc_hbm.at[srcg_sb.at[b]], asrc_blk, sem_c)
            cp_ad = pltpu.async_copy(adst_hbm.at[dst_sb.at[b]], adst_blk, sem_c)
            cp.wait()
            if b + 1 < SB:
                cp = nxt

            pltpu.sync_copy(rows_v, msg_hbm.at[midx_sb.at[b]])
            cp_as.wait()
            cp_ad.wait()

            for g in range(8):
                e16 = lane + g * 16
                z = plsc.load_gather(adst_blk, [e16, zeros16i]) + plsc.load_gather(
                    asrc_blk, [e16, zeros16i]
                )
                z = jnp.where(z >= 0.0, z, z * jnp.float32(0.01))
                w_sb[b, pl.ds(g * 16, 16)] = jnp.exp(z)

            brow = jnp.full((16,), b, jnp.int32)

            @plsc.parallel_loop(0, BLK, 1, unroll=4)
            def _edge(e):
                wb = plsc.load_gather(w_sb, [brow, jnp.full((16,), e, jnp.int32)])
                for v in range(DH // 16):
                    rows_v[e, pl.ds(v * 16, 16)] = (
                        rows_v[e, pl.ds(v * 16, 16)] * wb
                    )
                wrow_v[e, :] = jnp.where(lane == 0, wb, jnp.float32(0.0))

            pltpu.sync_copy(rows_v, agg_sh.at[dst_sb.at[b]], add=True)
            pltpu.sync_copy(wrow_v, den_sh.at[dst_sb.at[b]], add=True)

        pltpu.sync_copy(w_sb, w_hbm.at[s, pl.ds(k * SB, SB)])
        return carry

    lax.fori_loop(0, NSUP, _super, 0)

    plsc.subcore_barrier()
    for b in range(ROWS_PER_TILE // BLK):
        r0 = s * ROWS_PER_TILE + b * BLK
        pltpu.sync_copy(agg_sh.at[pl.ds(r0, BLK)], rows_a)
        pltpu.sync_copy(rows_a, aggp_hbm.at[pl.ds(r0, BLK), c])
        pltpu.sync_copy(den_sh.at[pl.ds(r0, BLK)], wrow_v)
        pltpu.sync_copy(wrow_v, denp_hbm.at[pl.ds(r0, BLK)])


_sc_call = functools.partial(
    pl.kernel,
    out_type=[
        jax.ShapeDtypeStruct((2 * E_REAL, DH), jnp.float32),
        jax.ShapeDtypeStruct((NS, NBLK, BLK), jnp.float32),
        jax.ShapeDtypeStruct((NPAD, 2, DH), jnp.float32),
        jax.ShapeDtypeStruct((NPAD, 16), jnp.float32),
    ],
    mesh=plsc.VectorSubcoreMesh(
        core_axis_name="c", subcore_axis_name="s", num_cores=NC, num_subcores=NS
    ),
    compiler_params=pltpu.CompilerParams(
        use_tc_tiling_on_sc=False, needs_layout_passes=False
    ),
    scratch_types=[
        pltpu.VMEM((SB, BLK), jnp.int32),
        pltpu.VMEM((SB, BLK), jnp.int32),
        pltpu.VMEM((SB, BLK), jnp.int32),
        pltpu.VMEM((SB, BLK), jnp.float32),
        pltpu.VMEM((BLK, DH), jnp.float32),
        pltpu.VMEM((BLK, DH), jnp.float32),
        pltpu.VMEM((BLK, 16), jnp.float32),
        pltpu.VMEM((BLK, 16), jnp.float32),
        pltpu.VMEM((BLK, 16), jnp.float32),
        pltpu.VMEM_SHARED((NPAD, DH), jnp.float32),
        pltpu.VMEM_SHARED((NPAD, 16), jnp.float32),
        pltpu.SemaphoreType.DMA,
        pltpu.SemaphoreType.DMA,
        pltpu.SemaphoreType.DMA,
    ],
)


def _comb_body(aggp_ref, denp_ref, agg_ref, den_ref):
    d = denp_ref[:, 0]
    safe = jnp.where(d == 0.0, jnp.float32(1.0), d)
    agg_ref[...] = aggp_ref[...] / safe[:, None]
    den_ref[...] = safe[:, None]


def _combine(aggp, denp):
    blk = 400
    return pl.pallas_call(
        _comb_body,
        grid=(N_NODES // blk,),
        in_specs=[
            pl.BlockSpec((blk, D), lambda i: (i, 0)),
            pl.BlockSpec((blk, 16), lambda i: (i, 0)),
        ],
        out_specs=[
            pl.BlockSpec((blk, D), lambda i: (i, 0)),
            pl.BlockSpec((blk, 1), lambda i: (i, 0)),
        ],
        out_shape=[
            jax.ShapeDtypeStruct((N_NODES, D), jnp.float32),
            jax.ShapeDtypeStruct((N_NODES, 1), jnp.float32),
        ],
    )(aggp, denp)


def kernel(x, edge_index, W, a):
    a2 = jnp.zeros((D, 8), jnp.float32)
    a2 = a2.at[:, 0].set(a[:D]).at[:, 1].set(a[D:])
    act2, asrc2, adst16 = _matmul(x, W.T, a2)
    glh = act2.reshape(2 * NPAD, DH)
    asrc16 = asrc2.reshape(2 * NPAD, 16)

    src = edge_index[0].astype(jnp.int32)
    dst = edge_index[1].astype(jnp.int32)
    loop = jnp.arange(N_NODES, dtype=jnp.int32)
    npad_e = EPAD - E_REAL
    src_full = jnp.concatenate(
        [src, loop, jnp.full((npad_e,), N_NODES - 1, jnp.int32)]
    )
    dst_full = jnp.concatenate([dst, loop, jnp.full((npad_e,), DUMMY, jnp.int32)])
    mpos = jnp.minimum(jnp.arange(EPAD, dtype=jnp.int32), E_REAL - 1) * 2
    src3d = src_full.reshape(NS, NBLK, BLK)
    dst3d = dst_full.reshape(NS, NBLK, BLK)
    mpos3d = mpos.reshape(NS, NBLK, BLK)

    msgf, w3d, aggp3, denp = _sc_call(_sc_body)(
        glh, src3d, dst3d, mpos3d, asrc16, adst16
    )
    agg, den = _combine(aggp3.reshape(NPAD, D), denp)

    return (
        agg,
        w3d.reshape(EPAD)[:E_REAL],
        den.reshape(N_NODES),
        msgf.reshape(E_REAL, D),
    )


# speedup vs baseline: 5.8781x; 1.0006x over previous
import functools

import jax
import jax.numpy as jnp
from jax import lax
from jax.experimental import pallas as pl
from jax.experimental.pallas import tpu as pltpu
from jax.experimental.pallas import tpu_sc as plsc

N_NODES = 10000
N_EDGES = 320000
D = 128
DH = D // 2
E_REAL = N_EDGES + N_NODES

NC, NS = 2, 16
NPAD = 10240
ROWS_PER_TILE = NPAD // NS
BLK = 128
SB = 6
NSUP = 27
NBLK = SB * NSUP
E_CHUNK = NBLK * BLK
EPAD = NS * E_CHUNK
DUMMY = N_NODES


def _mm_body(x_ref, wt_ref, a2_ref, act2_ref, asrc_ref, adst_ref):
    act = jnp.dot(x_ref[...], wt_ref[...], preferred_element_type=jnp.float32)
    a2 = a2_ref[...]
    adst = jnp.sum(act * a2[:, 0][None, :], axis=1)
    asrc = jnp.sum(act * a2[:, 1][None, :], axis=1)
    lane = lax.broadcasted_iota(jnp.int32, (act.shape[0], 16), 1)
    asrc16 = jnp.where(lane == 0, asrc[:, None], 0.0)
    act2_ref[0] = act[:, :DH]
    act2_ref[1] = act[:, DH:]
    asrc_ref[0] = asrc16
    asrc_ref[1] = asrc16
    adst_ref[...] = jnp.where(lane == 0, adst[:, None], 0.0)


def _matmul(x, wt, a2):
    blk = 400
    return pl.pallas_call(
        _mm_body,
        grid=(N_NODES // blk,),
        in_specs=[
            pl.BlockSpec((blk, D), lambda i: (i, 0)),
            pl.BlockSpec((D, D), lambda i: (0, 0)),
            pl.BlockSpec((D, 8), lambda i: (0, 0)),
        ],
        out_specs=[
            pl.BlockSpec((2, blk, DH), lambda i: (0, i, 0)),
            pl.BlockSpec((2, blk, 16), lambda i: (0, i, 0)),
            pl.BlockSpec((blk, 16), lambda i: (i, 0)),
        ],
        out_shape=[
            jax.ShapeDtypeStruct((2, NPAD, DH), jnp.float32),
            jax.ShapeDtypeStruct((2, NPAD, 16), jnp.float32),
            jax.ShapeDtypeStruct((NPAD, 16), jnp.float32),
        ],
    )(x, wt, a2)


def _sc_body(glh_hbm, src_hbm, dst_hbm, mpos_hbm, asrc_hbm, adst_hbm,
             msg_hbm, w_hbm, aggp_hbm, denp_hbm,
             srcg_sb, dst_sb, midx_sb, w_sb, rows_a, rows_b, wrow_v,
             asrc_blk, adst_blk, agg_sh, den_sh, sem_a, sem_b, sem_c):
    c = lax.axis_index("c")
    s = lax.axis_index("s")

    def _zero_row(e, carry):
        for v in range(DH // 16):
            rows_a[e, pl.ds(v * 16, 16)] = jnp.zeros((16,), jnp.float32)
        wrow_v[e, :] = jnp.zeros((16,), jnp.float32)
        return carry

    lax.fori_loop(0, BLK, _zero_row, 0)
    for b in range(ROWS_PER_TILE // BLK):
        r0 = s * ROWS_PER_TILE + b * BLK
        pltpu.sync_copy(rows_a, agg_sh.at[pl.ds(r0, BLK)])
        pltpu.sync_copy(wrow_v, den_sh.at[pl.ds(r0, BLK)])
    plsc.subcore_barrier()

    lane = lax.iota(jnp.int32, 16)
    zeros16i = jnp.zeros((16,), jnp.int32)
    coff = jnp.full((16,), c, jnp.int32) * jnp.int32(NPAD)
    cvec = jnp.full((16,), c, jnp.int32)
    bufs = ((rows_a, sem_a), (rows_b, sem_b))

    def _fire(b, rows_v, sem):
        return pltpu.async_copy(glh_hbm.at[srcg_sb.at[b]], rows_v, sem)

    def _super(k, carry):
        pltpu.sync_copy(src_hbm.at[s, pl.ds(k * SB, SB)], srcg_sb)
        pltpu.sync_copy(dst_hbm.at[s, pl.ds(k * SB, SB)], dst_sb)
        pltpu.sync_copy(mpos_hbm.at[s, pl.ds(k * SB, SB)], midx_sb)
        for b in range(SB):
            for g in range(8):
                srcg_sb[b, pl.ds(g * 16, 16)] = (
                    srcg_sb[b, pl.ds(g * 16, 16)] + coff
                )
                midx_sb[b, pl.ds(g * 16, 16)] = (
                    midx_sb[b, pl.ds(g * 16, 16)] + cvec
                )

        cp = _fire(0, *bufs[0])
        for b in range(SB):
            rows_v, _ = bufs[b % 2]
            if b + 1 < SB:
                nxt = _fire(b + 1, *bufs[(b + 1) % 2])
            cp_as = pltpu.async_copy(asrc_hbm.at[srcg_sb.at[b]], asrc_blk, sem_c)
            cp_ad = pltpu.async_copy(adst_hbm.at[dst_sb.at[b]], adst_blk, sem_c)
            cp.wait()
            if b + 1 < SB:
                cp = nxt

            pltpu.sync_copy(rows_v, msg_hbm.at[midx_sb.at[b]])
            cp_as.wait()
            cp_ad.wait()

            for g in range(8):
                e16 = lane + g * 16
                z = plsc.load_gather(adst_blk, [e16, zeros16i]) + plsc.load_gather(
                    asrc_blk, [e16, zeros16i]
                )
                z = jnp.where(z >= 0.0, z, z * jnp.float32(0.01))
                w_sb[b, pl.ds(g * 16, 16)] = jnp.exp(z)

            brow = jnp.full((16,), b, jnp.int32)

            @plsc.parallel_loop(0, BLK, 1, unroll=4)
            def _edge(e):
                wb = plsc.load_gather(w_sb, [brow, jnp.full((16,), e, jnp.int32)])
                for v in range(DH // 16):
                    rows_v[e, pl.ds(v * 16, 16)] = (
                        rows_v[e, pl.ds(v * 16, 16)] * wb
                    )
                wrow_v[e, :] = jnp.where(lane == 0, wb, jnp.float32(0.0))

            pltpu.sync_copy(rows_v, agg_sh.at[dst_sb.at[b]], add=True)
            pltpu.sync_copy(wrow_v, den_sh.at[dst_sb.at[b]], add=True)

        pltpu.sync_copy(w_sb, w_hbm.at[s, pl.ds(k * SB, SB)])
        return carry

    lax.fori_loop(0, NSUP, _super, 0)

    plsc.subcore_barrier()
    for b in range(ROWS_PER_TILE // BLK):
        r0 = s * ROWS_PER_TILE + b * BLK
        pltpu.sync_copy(agg_sh.at[pl.ds(r0, BLK)], rows_a)
        pltpu.sync_copy(rows_a, aggp_hbm.at[pl.ds(r0, BLK), c])
        pltpu.sync_copy(den_sh.at[pl.ds(r0, BLK)], wrow_v)
        pltpu.sync_copy(wrow_v, denp_hbm.at[pl.ds(r0, BLK)])


_sc_call = functools.partial(
    pl.kernel,
    out_type=[
        jax.ShapeDtypeStruct((2 * E_REAL, DH), jnp.float32),
        jax.ShapeDtypeStruct((NS, NBLK, BLK), jnp.float32),
        jax.ShapeDtypeStruct((NPAD, 2, DH), jnp.float32),
        jax.ShapeDtypeStruct((NPAD, 16), jnp.float32),
    ],
    mesh=plsc.VectorSubcoreMesh(
        core_axis_name="c", subcore_axis_name="s", num_cores=NC, num_subcores=NS
    ),
    compiler_params=pltpu.CompilerParams(
        use_tc_tiling_on_sc=False, needs_layout_passes=False
    ),
    scratch_types=[
        pltpu.VMEM((SB, BLK), jnp.int32),
        pltpu.VMEM((SB, BLK), jnp.int32),
        pltpu.VMEM((SB, BLK), jnp.int32),
        pltpu.VMEM((SB, BLK), jnp.float32),
        pltpu.VMEM((BLK, DH), jnp.float32),
        pltpu.VMEM((BLK, DH), jnp.float32),
        pltpu.VMEM((BLK, 16), jnp.float32),
        pltpu.VMEM((BLK, 16), jnp.float32),
        pltpu.VMEM((BLK, 16), jnp.float32),
        pltpu.VMEM_SHARED((NPAD, DH), jnp.float32),
        pltpu.VMEM_SHARED((NPAD, 16), jnp.float32),
        pltpu.SemaphoreType.DMA,
        pltpu.SemaphoreType.DMA,
        pltpu.SemaphoreType.DMA,
    ],
)


def _comb_body(aggp_ref, denp_ref, agg_ref, den_ref):
    d = denp_ref[:, 0]
    safe = jnp.where(d == 0.0, jnp.float32(1.0), d)
    agg_ref[...] = aggp_ref[...] / safe[:, None]
    den_ref[...] = safe[:, None]


def _combine(aggp, denp):
    blk = 400
    return pl.pallas_call(
        _comb_body,
        grid=(N_NODES // blk,),
        in_specs=[
            pl.BlockSpec((blk, D), lambda i: (i, 0)),
            pl.BlockSpec((blk, 16), lambda i: (i, 0)),
        ],
        out_specs=[
            pl.BlockSpec((blk, D), lambda i: (i, 0)),
            pl.BlockSpec((blk, 1), lambda i: (i, 0)),
        ],
        out_shape=[
            jax.ShapeDtypeStruct((N_NODES, D), jnp.float32),
            jax.ShapeDtypeStruct((N_NODES, 1), jnp.float32),
        ],
    )(aggp, denp)


def kernel(x, edge_index, W, a):
    a2 = jnp.zeros((D, 8), jnp.float32)
    a2 = a2.at[:, 0].set(a[:D]).at[:, 1].set(a[D:])
    act2, asrc2, adst16 = _matmul(x, W.T, a2)
    glh = act2.reshape(2 * NPAD, DH)
    asrc16 = asrc2.reshape(2 * NPAD, 16)

    src = edge_index[0].astype(jnp.int32)
    dst = edge_index[1].astype(jnp.int32)
    loop = jnp.arange(N_NODES, dtype=jnp.int32)
    npad_e = EPAD - E_REAL
    src_full = jnp.concatenate(
        [src, loop, jnp.full((npad_e,), N_NODES - 1, jnp.int32)]
    )
    dst_full = jnp.concatenate([dst, loop, jnp.full((npad_e,), DUMMY, jnp.int32)])
    mpos = jnp.minimum(jnp.arange(EPAD, dtype=jnp.int32), E_REAL - 1) * 2
    src3d = src_full.reshape(NS, NBLK, BLK)
    dst3d = dst_full.reshape(NS, NBLK, BLK)
    mpos3d = mpos.reshape(NS, NBLK, BLK)

    msgf, w3d, aggp3, denp = _sc_call(_sc_body)(
        glh, src3d, dst3d, mpos3d, asrc16, adst16
    )
    agg, den = _combine(aggp3.reshape(NPAD, D), denp)

    return (
        agg,
        w3d.reshape(EPAD)[:E_REAL],
        den.reshape(N_NODES),
        msgf.reshape(E_REAL, D),
    )
